# baseline (speedup 1.0000x reference)
import functools

import jax
import jax.numpy as jnp
from jax import lax
from jax.experimental import pallas as pl
from jax.experimental.pallas import tpu as pltpu
from jax.experimental.pallas import tpu_sc as plsc

_N = 10000
_NP = 10240
_D = 128
_E = 320000
_NC, _NS = 2, 16
_NW = _NC * _NS
_CHUNK = 128
_EP = 327680
_CH = _EP // (_NW * _CHUNK)
_RPT = _NP // _NS
_PAD_IDX = _N

_mesh = plsc.VectorSubcoreMesh(core_axis_name="c", subcore_axis_name="s")


@functools.partial(
    pl.kernel,
    out_type=(
        jax.ShapeDtypeStruct((_NC, _NP, _D), jnp.float32),
        jax.ShapeDtypeStruct((_NC, _NP, _D), jnp.float32),
    ),
    mesh=_mesh,
    scratch_types=[
        pltpu.VMEM((_CH, _CHUNK), jnp.int32),
        pltpu.VMEM((_CH, _CHUNK), jnp.int32),
        pltpu.VMEM((_CHUNK, _D), jnp.float32),
        pltpu.VMEM_SHARED((_NP, _D), jnp.float32),
        pltpu.SemaphoreType.DMA,
    ],
)
def _sc_degrees(src_hbm, dst_hbm, z_hbm,
                dego_out, degi_out,
                src_v, dst_v, ones_v, acc_sh, dsem):
    c = lax.axis_index("c")
    s = lax.axis_index("s")
    wid = c * _NS + s
    pltpu.sync_copy(src_hbm.at[pl.ds(wid * _CH, _CH)], src_v)
    pltpu.sync_copy(dst_hbm.at[pl.ds(wid * _CH, _CH)], dst_v)

    def fill(r, carry):
        for l in range(_D // 16):
            ones_v[r, pl.ds(l * 16, 16)] = jnp.ones((16,), jnp.float32)
        return carry

    lax.fori_loop(0, _CHUNK, fill, 0)

    def one_table(idx_v, out_ref):
        pltpu.sync_copy(z_hbm.at[pl.ds(s * _RPT, _RPT)],
                        acc_sh.at[pl.ds(s * _RPT, _RPT)])
        plsc.subcore_barrier()

        def body(g, carry):
            base = g * 8
            for k in range(8):
                pltpu.async_copy(ones_v, acc_sh.at[idx_v.at[base + k]],
                                 dsem, add=True)
            for k in range(8):
                pltpu.make_async_copy(ones_v, acc_sh.at[idx_v.at[base + k]],
                                      dsem).wait()
            return carry

        lax.fori_loop(0, _CH // 8, body, 0)
        plsc.subcore_barrier()
        pltpu.sync_copy(acc_sh.at[pl.ds(s * _RPT, _RPT)],
                        out_ref.at[c, pl.ds(s * _RPT, _RPT)])
        plsc.subcore_barrier()

    one_table(src_v, dego_out)
    one_table(dst_v, degi_out)


_NBUF = 2
_PH0 = 4
_CH0 = _CH * _NC
_CHPMAX = _CH0 // _PH0


@functools.partial(
    pl.kernel,
    out_type=jax.ShapeDtypeStruct((_NP, _D), jnp.float32),
    mesh=_mesh,
    scratch_types=[
        pltpu.VMEM((_CHPMAX, _CHUNK), jnp.int32),
        pltpu.VMEM((_CHPMAX, _CHUNK), jnp.int32),
    ]
    + [pltpu.VMEM((_CHUNK, _D), jnp.float32)] * _NBUF
    + [pltpu.VMEM_SHARED((_NP, _D), jnp.float32)]
    + [pltpu.SemaphoreType.DMA] * (3 * _NBUF),
)
def _sc_aggregate(h_hbm, src_hbm, dst_hbm, z_hbm, agg_out,
                  src_v, dst_v, r0, r1, acc_sh, *sems):
    rows_v = (r0, r1)
    gsems = (sems[0:2], sems[2:4])
    ssems = sems[4:]
    _H = _CHUNK // 2
    c = lax.axis_index("c")
    s = lax.axis_index("s")

    @pl.when(c == 0)
    def _():
        pltpu.sync_copy(z_hbm.at[pl.ds(s * _RPT, _RPT)],
                        acc_sh.at[pl.ds(s * _RPT, _RPT)])

    plsc.subcore_barrier()

    def run_core(tile_base, n_chunks, phases):
        chp = n_chunks // phases
        for p in range(phases):
            base = tile_base + p * chp
            pltpu.sync_copy(src_hbm.at[pl.ds(base, chp)],
                            src_v.at[pl.ds(0, chp)])
            pltpu.sync_copy(dst_hbm.at[pl.ds(base, chp)],
                            dst_v.at[pl.ds(0, chp)])
            def gather(ci, b):
                for hh in range(2):
                    pltpu.async_copy(
                        h_hbm.at[src_v.at[ci, pl.ds(hh * _H, _H)]],
                        rows_v[b].at[pl.ds(hh * _H, _H)], gsems[b][hh])

            def gather_wait(ci, b):
                for hh in range(2):
                    pltpu.make_async_copy(
                        h_hbm.at[src_v.at[ci, pl.ds(hh * _H, _H)]],
                        rows_v[b].at[pl.ds(hh * _H, _H)], gsems[b][hh]).wait()

            for b in range(_NBUF):
                gather(b, b)

            def body(jj, carry):
                for b in range(_NBUF):
                    ci = jj * _NBUF + b
                    gather_wait(ci, b)
                    pltpu.async_copy(
                        rows_v[b], acc_sh.at[dst_v.at[ci]], ssems[b],
                        add=True)
                    pltpu.make_async_copy(
                        rows_v[b], acc_sh.at[dst_v.at[ci]], ssems[b]).wait()

                    @pl.when(ci + _NBUF < chp)
                    def _():
                        gather(ci + _NBUF, b)

                return carry

            lax.fori_loop(0, chp // _NBUF, body, 0)

    @pl.when(c == 0)
    def _():
        run_core(s * _CH0, _CH0, _PH0)

    plsc.subcore_barrier()

    @pl.when(c == 0)
    def _():
        pltpu.sync_copy(acc_sh.at[pl.ds(s * _RPT, _RPT)],
                        agg_out.at[pl.ds(s * _RPT, _RPT)])


_BR = 256


def _norm_body(dego_ref, degi_ref, x_ref, xs_ref, nsrc_ref, ndst_ref):
    do = dego_ref[0] + dego_ref[1]
    di = degi_ref[0] + degi_ref[1]
    nsrc_b = lax.rsqrt(jnp.maximum(do, 1.0))
    ndst_b = lax.rsqrt(jnp.maximum(di, 1.0))
    xs_ref[...] = x_ref[...] * nsrc_b
    nsrc_ref[...] = nsrc_b
    ndst_ref[...] = ndst_b


_tc_norms = pl.pallas_call(
    _norm_body,
    grid=(_NP // _BR,),
    in_specs=[
        pl.BlockSpec((_NC, _BR, _D), lambda i: (0, i, 0)),
        pl.BlockSpec((_NC, _BR, _D), lambda i: (0, i, 0)),
        pl.BlockSpec((_BR, _D), lambda i: (i, 0)),
    ],
    out_specs=[
        pl.BlockSpec((_BR, _D), lambda i: (i, 0)),
        pl.BlockSpec((_BR, _D), lambda i: (i, 0)),
        pl.BlockSpec((_BR, _D), lambda i: (i, 0)),
    ],
    out_shape=[jax.ShapeDtypeStruct((_NP, _D), jnp.float32)] * 3,
)


def _layer_body(agg_ref, ndst_ref, w_ref, b_ref, nsrc_ref, o_ref):
    agg = agg_ref[...] * ndst_ref[...]
    out = jnp.dot(agg, w_ref[...], preferred_element_type=jnp.float32)
    out = out + b_ref[...]
    act = jnp.where(out > 0.0, out, jnp.exp(jnp.minimum(out, 0.0)) - 1.0)
    o_ref[...] = act * nsrc_ref[...]


_tc_layer = pl.pallas_call(
    _layer_body,
    grid=(_NP // _BR,),
    in_specs=[
        pl.BlockSpec((_BR, _D), lambda i: (i, 0)),
        pl.BlockSpec((_BR, _D), lambda i: (i, 0)),
        pl.BlockSpec((_D, _D), lambda i: (0, 0)),
        pl.BlockSpec((1, _D), lambda i: (0, 0)),
        pl.BlockSpec((_BR, _D), lambda i: (i, 0)),
    ],
    out_specs=pl.BlockSpec((_BR, _D), lambda i: (i, 0)),
    out_shape=jax.ShapeDtypeStruct((_NP, _D), jnp.float32),
)


def _final_body(agg_ref, ndst_ref, w2_ref, b2_ref, wo_ref, bo_ref,
                emb_ref, out_ref):
    agg = agg_ref[...] * ndst_ref[...]
    emb = jnp.dot(agg, w2_ref[...], preferred_element_type=jnp.float32)
    emb = emb + b2_ref[...]
    emb_ref[...] = emb
    out = jnp.dot(emb, wo_ref[...], preferred_element_type=jnp.float32)
    out_ref[...] = out + bo_ref[...]


_tc_final = pl.pallas_call(
    _final_body,
    grid=(_NP // _BR,),
    in_specs=[
        pl.BlockSpec((_BR, _D), lambda i: (i, 0)),
        pl.BlockSpec((_BR, _D), lambda i: (i, 0)),
        pl.BlockSpec((_D, _D), lambda i: (0, 0)),
        pl.BlockSpec((1, _D), lambda i: (0, 0)),
        pl.BlockSpec((_D, _D), lambda i: (0, 0)),
        pl.BlockSpec((1, _D), lambda i: (0, 0)),
    ],
    out_specs=[
        pl.BlockSpec((_BR, _D), lambda i: (i, 0)),
        pl.BlockSpec((_BR, _D), lambda i: (i, 0)),
    ],
    out_shape=[
        jax.ShapeDtypeStruct((_NP, _D), jnp.float32),
        jax.ShapeDtypeStruct((_NP, _D), jnp.float32),
    ],
)


def kernel(x, edge_index, W0, b0, W1, b1, W2, b2, W_out, b_out):
    xp = jnp.zeros((_NP, _D), jnp.float32).at[:_N].set(x)
    pad = jnp.full((_EP - _E,), _PAD_IDX, jnp.int32)
    src = jnp.concatenate([edge_index[0], pad]).reshape(_NW * _CH, _CHUNK)
    dst = jnp.concatenate([edge_index[1], pad]).reshape(_NW * _CH, _CHUNK)
    z = jnp.zeros((_NP, _D), jnp.float32)
    b0r, b1r, b2r = b0.reshape(1, _D), b1.reshape(1, _D), b2.reshape(1, _D)
    wo = jnp.zeros((_D, _D), jnp.float32).at[:, : W_out.shape[1]].set(W_out)
    bo = jnp.zeros((1, _D), jnp.float32).at[0, : b_out.shape[0]].set(b_out)

    dego_p, degi_p = _sc_degrees(src, dst, z)
    h0, nsrc_b, ndst_b = _tc_norms(dego_p, degi_p, xp)

    agg0 = _sc_aggregate(h0, src, dst, z)
    h1 = _tc_layer(agg0, ndst_b, W0, b0r, nsrc_b)
    agg1 = _sc_aggregate(h1, src, dst, z)
    h2 = _tc_layer(agg1, ndst_b, W1, b1r, nsrc_b)
    agg2 = _sc_aggregate(h2, src, dst, z)
    n_embed_p, n_out_p = _tc_final(agg2, ndst_b, W2, b2r, wo, bo)

    n_embed = n_embed_p[:_N]
    n_out = n_out_p[:_N, : W_out.shape[1]]
    return (n_out, n_embed)


# speedup vs baseline: 1.3862x; 1.3862x over previous
import functools

import jax
import jax.numpy as jnp
from jax import lax
from jax.experimental import pallas as pl
from jax.experimental.pallas import tpu as pltpu
from jax.experimental.pallas import tpu_sc as plsc

_N = 10000
_NP = 10240
_D = 128
_E = 320000
_NC, _NS = 2, 16
_NW = _NC * _NS
_CHUNK = 128
_EP = 327680
_CH = _EP // (_NW * _CHUNK)
_RPT = _NP // _NS
_PAD_IDX = _N

_mesh = plsc.VectorSubcoreMesh(core_axis_name="c", subcore_axis_name="s")


@functools.partial(
    pl.kernel,
    out_type=(
        jax.ShapeDtypeStruct((_NC, _NP, _D), jnp.float32),
        jax.ShapeDtypeStruct((_NC, _NP, _D), jnp.float32),
    ),
    mesh=_mesh,
    scratch_types=[
        pltpu.VMEM((_CH, _CHUNK), jnp.int32),
        pltpu.VMEM((_CH, _CHUNK), jnp.int32),
        pltpu.VMEM((_CHUNK, _D), jnp.float32),
        pltpu.VMEM_SHARED((_NP, _D), jnp.float32),
        pltpu.SemaphoreType.DMA,
    ],
)
def _sc_degrees(src_hbm, dst_hbm, z_hbm,
                dego_out, degi_out,
                src_v, dst_v, ones_v, acc_sh, dsem):
    c = lax.axis_index("c")
    s = lax.axis_index("s")
    wid = c * _NS + s
    pltpu.sync_copy(src_hbm.at[pl.ds(wid * _CH, _CH)], src_v)
    pltpu.sync_copy(dst_hbm.at[pl.ds(wid * _CH, _CH)], dst_v)

    def fill(r, carry):
        for l in range(_D // 16):
            ones_v[r, pl.ds(l * 16, 16)] = jnp.ones((16,), jnp.float32)
        return carry

    lax.fori_loop(0, _CHUNK, fill, 0)

    def one_table(idx_v, out_ref):
        pltpu.sync_copy(z_hbm.at[pl.ds(s * _RPT, _RPT)],
                        acc_sh.at[pl.ds(s * _RPT, _RPT)])
        plsc.subcore_barrier()

        def body(g, carry):
            base = g * 8
            for k in range(8):
                pltpu.async_copy(ones_v, acc_sh.at[idx_v.at[base + k]],
                                 dsem, add=True)
            for k in range(8):
                pltpu.make_async_copy(ones_v, acc_sh.at[idx_v.at[base + k]],
                                      dsem).wait()
            return carry

        lax.fori_loop(0, _CH // 8, body, 0)
        plsc.subcore_barrier()
        pltpu.sync_copy(acc_sh.at[pl.ds(s * _RPT, _RPT)],
                        out_ref.at[c, pl.ds(s * _RPT, _RPT)])
        plsc.subcore_barrier()

    one_table(src_v, dego_out)
    one_table(dst_v, degi_out)


_NBUF = 2
_PH0, _PH1 = 3, 2
_CH0 = 144
_CH1 = _CH * _NC - _CH0
_CHPMAX = max(_CH0 // _PH0, _CH1 // _PH1)


@functools.partial(
    pl.kernel,
    out_type=jax.ShapeDtypeStruct((_NC, _NP, _D), jnp.float32),
    mesh=_mesh,
    scratch_types=[
        pltpu.VMEM((_CHPMAX, _CHUNK), jnp.int32),
        pltpu.VMEM((_CHPMAX, _CHUNK), jnp.int32),
    ]
    + [pltpu.VMEM((_CHUNK, _D), jnp.float32)] * _NBUF
    + [pltpu.VMEM_SHARED((_NP, _D), jnp.float32)]
    + [pltpu.SemaphoreType.DMA] * (3 * _NBUF),
)
def _sc_aggregate(h_hbm, src_hbm, dst_hbm, z_hbm, agg_out,
                  src_v, dst_v, r0, r1, acc_sh, *sems):
    rows_v = (r0, r1)
    gsems = (sems[0:2], sems[2:4])
    ssems = sems[4:]
    _H = _CHUNK // 2
    c = lax.axis_index("c")
    s = lax.axis_index("s")
    pltpu.sync_copy(z_hbm.at[pl.ds(s * _RPT, _RPT)],
                    acc_sh.at[pl.ds(s * _RPT, _RPT)])
    plsc.subcore_barrier()

    def run_core(tile_base, n_chunks, phases):
        chp = n_chunks // phases
        for p in range(phases):
            base = tile_base + p * chp
            pltpu.sync_copy(src_hbm.at[pl.ds(base, chp)],
                            src_v.at[pl.ds(0, chp)])
            pltpu.sync_copy(dst_hbm.at[pl.ds(base, chp)],
                            dst_v.at[pl.ds(0, chp)])
            def gather(ci, b):
                for hh in range(2):
                    pltpu.async_copy(
                        h_hbm.at[src_v.at[ci, pl.ds(hh * _H, _H)]],
                        rows_v[b].at[pl.ds(hh * _H, _H)], gsems[b][hh])

            def gather_wait(ci, b):
                for hh in range(2):
                    pltpu.make_async_copy(
                        h_hbm.at[src_v.at[ci, pl.ds(hh * _H, _H)]],
                        rows_v[b].at[pl.ds(hh * _H, _H)], gsems[b][hh]).wait()

            for b in range(_NBUF):
                gather(b, b)

            def body(jj, carry):
                for b in range(_NBUF):
                    ci = jj * _NBUF + b
                    gather_wait(ci, b)
                    pltpu.async_copy(
                        rows_v[b], acc_sh.at[dst_v.at[ci]], ssems[b],
                        add=True)
                    pltpu.make_async_copy(
                        rows_v[b], acc_sh.at[dst_v.at[ci]], ssems[b]).wait()

                    @pl.when(ci + _NBUF < chp)
                    def _():
                        gather(ci + _NBUF, b)

                return carry

            lax.fori_loop(0, chp // _NBUF, body, 0)

    @pl.when(c == 0)
    def _():
        run_core(s * _CH0, _CH0, _PH0)

    @pl.when(c == 1)
    def _():
        run_core(_NS * _CH0 + s * _CH1, _CH1, _PH1)

    plsc.subcore_barrier()
    pltpu.sync_copy(acc_sh.at[pl.ds(s * _RPT, _RPT)],
                    agg_out.at[c, pl.ds(s * _RPT, _RPT)])


_BR = 256


def _norm_body(dego_ref, degi_ref, x_ref, xs_ref, nsrc_ref, ndst_ref):
    do = dego_ref[0] + dego_ref[1]
    di = degi_ref[0] + degi_ref[1]
    nsrc_b = lax.rsqrt(jnp.maximum(do, 1.0))
    ndst_b = lax.rsqrt(jnp.maximum(di, 1.0))
    xs_ref[...] = x_ref[...] * nsrc_b
    nsrc_ref[...] = nsrc_b
    ndst_ref[...] = ndst_b


_tc_norms = pl.pallas_call(
    _norm_body,
    grid=(_NP // _BR,),
    in_specs=[
        pl.BlockSpec((_NC, _BR, _D), lambda i: (0, i, 0)),
        pl.BlockSpec((_NC, _BR, _D), lambda i: (0, i, 0)),
        pl.BlockSpec((_BR, _D), lambda i: (i, 0)),
    ],
    out_specs=[
        pl.BlockSpec((_BR, _D), lambda i: (i, 0)),
        pl.BlockSpec((_BR, _D), lambda i: (i, 0)),
        pl.BlockSpec((_BR, _D), lambda i: (i, 0)),
    ],
    out_shape=[jax.ShapeDtypeStruct((_NP, _D), jnp.float32)] * 3,
)


def _layer_body(agg_ref, ndst_ref, w_ref, b_ref, nsrc_ref, o_ref):
    agg = (agg_ref[0] + agg_ref[1]) * ndst_ref[...]
    out = jnp.dot(agg, w_ref[...], preferred_element_type=jnp.float32)
    out = out + b_ref[...]
    act = jnp.where(out > 0.0, out, jnp.exp(jnp.minimum(out, 0.0)) - 1.0)
    o_ref[...] = act * nsrc_ref[...]


_tc_layer = pl.pallas_call(
    _layer_body,
    grid=(_NP // _BR,),
    in_specs=[
        pl.BlockSpec((_NC, _BR, _D), lambda i: (0, i, 0)),
        pl.BlockSpec((_BR, _D), lambda i: (i, 0)),
        pl.BlockSpec((_D, _D), lambda i: (0, 0)),
        pl.BlockSpec((1, _D), lambda i: (0, 0)),
        pl.BlockSpec((_BR, _D), lambda i: (i, 0)),
    ],
    out_specs=pl.BlockSpec((_BR, _D), lambda i: (i, 0)),
    out_shape=jax.ShapeDtypeStruct((_NP, _D), jnp.float32),
)


def _final_body(agg_ref, ndst_ref, w2_ref, b2_ref, wo_ref, bo_ref,
                emb_ref, out_ref):
    agg = (agg_ref[0] + agg_ref[1]) * ndst_ref[...]
    emb = jnp.dot(agg, w2_ref[...], preferred_element_type=jnp.float32)
    emb = emb + b2_ref[...]
    emb_ref[...] = emb
    out = jnp.dot(emb, wo_ref[...], preferred_element_type=jnp.float32)
    out_ref[...] = out + bo_ref[...]


_tc_final = pl.pallas_call(
    _final_body,
    grid=(_NP // _BR,),
    in_specs=[
        pl.BlockSpec((_NC, _BR, _D), lambda i: (0, i, 0)),
        pl.BlockSpec((_BR, _D), lambda i: (i, 0)),
        pl.BlockSpec((_D, _D), lambda i: (0, 0)),
        pl.BlockSpec((1, _D), lambda i: (0, 0)),
        pl.BlockSpec((_D, _D), lambda i: (0, 0)),
        pl.BlockSpec((1, _D), lambda i: (0, 0)),
    ],
    out_specs=[
        pl.BlockSpec((_BR, _D), lambda i: (i, 0)),
        pl.BlockSpec((_BR, _D), lambda i: (i, 0)),
    ],
    out_shape=[
        jax.ShapeDtypeStruct((_NP, _D), jnp.float32),
        jax.ShapeDtypeStruct((_NP, _D), jnp.float32),
    ],
)


def kernel(x, edge_index, W0, b0, W1, b1, W2, b2, W_out, b_out):
    xp = jnp.zeros((_NP, _D), jnp.float32).at[:_N].set(x)
    pad = jnp.full((_EP - _E,), _PAD_IDX, jnp.int32)
    src = jnp.concatenate([edge_index[0], pad]).reshape(_NW * _CH, _CHUNK)
    dst = jnp.concatenate([edge_index[1], pad]).reshape(_NW * _CH, _CHUNK)
    z = jnp.zeros((_NP, _D), jnp.float32)
    b0r, b1r, b2r = b0.reshape(1, _D), b1.reshape(1, _D), b2.reshape(1, _D)
    wo = jnp.zeros((_D, _D), jnp.float32).at[:, : W_out.shape[1]].set(W_out)
    bo = jnp.zeros((1, _D), jnp.float32).at[0, : b_out.shape[0]].set(b_out)

    dego_p, degi_p = _sc_degrees(src, dst, z)
    h0, nsrc_b, ndst_b = _tc_norms(dego_p, degi_p, xp)

    agg0 = _sc_aggregate(h0, src, dst, z)
    h1 = _tc_layer(agg0, ndst_b, W0, b0r, nsrc_b)
    agg1 = _sc_aggregate(h1, src, dst, z)
    h2 = _tc_layer(agg1, ndst_b, W1, b1r, nsrc_b)
    agg2 = _sc_aggregate(h2, src, dst, z)
    n_embed_p, n_out_p = _tc_final(agg2, ndst_b, W2, b2r, wo, bo)

    n_embed = n_embed_p[:_N]
    n_out = n_out_p[:_N, : W_out.shape[1]]
    return (n_out, n_embed)
